# baseline (speedup 1.0000x reference)
import functools

import jax
import jax.numpy as jnp
from jax.experimental import pallas as pl
from jax.experimental.pallas import tpu as pltpu

_VMEM_LIMIT = 48 * 1024 * 1024


def _c1_kernel(x_ref, w_ref, b_ref, o_ref, *, rb):
    r2 = 2 * rb
    row0 = pl.program_id(1) * r2
    m = r2 * 112
    pieces = [x_ref[0, pl.ds(row0 + di, r2), :, :].reshape(m, 18)
              for di in range(5)]
    xm = jnp.concatenate(pieces, axis=-1)
    acc = jnp.dot(xm, w_ref[...], preferred_element_type=jnp.float32)
    cmax = jnp.maximum(acc[:, :64], acc[:, 64:])
    cmax = cmax.reshape(rb, 2, 112, 64)
    rmax = jnp.maximum(cmax[:, 0], cmax[:, 1])
    out = jnp.maximum(rmax + b_ref[...].reshape(1, 1, 64), 0.0)
    out = jnp.pad(out, ((0, 0), (0, 16), (0, 0)))
    col = jax.lax.broadcasted_iota(jnp.int32, (rb, 128, 64), 1)
    out = jnp.where(col < 110, out, 0.0)
    o_ref[...] = out.astype(o_ref.dtype).reshape(1, rb, 128, 64)


def _c3x3_kernel(x_ref, w_ref, b_ref, o_ref, *, rb, wbv):
    _, h, wp2, cin2 = x_ref.shape
    cout = o_ref.shape[-1]
    r2 = 2 * rb
    row0 = pl.program_id(1) * r2
    m = r2 * wp2
    pieces = []
    for di in range(3):
        win = x_ref[0, pl.ds(row0 + di, r2), :, :].reshape(m, cin2)
        pieces.append(win)
        pieces.append(jnp.pad(win[1:, :], ((0, 1), (0, 0))))
    xm = jnp.concatenate(pieces, axis=-1)
    acc = jnp.dot(xm, w_ref[...], preferred_element_type=jnp.float32)
    cmax = jnp.maximum(acc[:, :cout], acc[:, cout:])
    cmax = cmax.reshape(rb, 2, wp2, cout)
    rmax = jnp.maximum(cmax[:, 0], cmax[:, 1])
    out = jnp.maximum(rmax + b_ref[...].reshape(1, 1, cout), 0.0)
    col = jax.lax.broadcasted_iota(jnp.int32, (rb, wp2, cout), 1)
    out = jnp.where(col < wbv, out, 0.0)
    o_ref[...] = out.astype(o_ref.dtype).reshape(1, rb, wp2, cout)


def _conv1_pool(xp, w_eo, b, *, rb):
    n = xp.shape[0]
    return pl.pallas_call(
        functools.partial(_c1_kernel, rb=rb),
        out_shape=jax.ShapeDtypeStruct((n, 110, 128, 64), jnp.bfloat16),
        grid_spec=pltpu.PrefetchScalarGridSpec(
            num_scalar_prefetch=0,
            grid=(n, 110 // rb),
            in_specs=[
                pl.BlockSpec((1, 224, 112, 18), lambda i, r: (i, 0, 0, 0)),
                pl.BlockSpec((90, 128), lambda i, r: (0, 0)),
                pl.BlockSpec((1, 64), lambda i, r: (0, 0)),
            ],
            out_specs=pl.BlockSpec((1, rb, 128, 64), lambda i, r: (i, r, 0, 0)),
        ),
        compiler_params=pltpu.CompilerParams(
            dimension_semantics=("parallel", "arbitrary"),
            vmem_limit_bytes=_VMEM_LIMIT),
    )(xp, w_eo, b)


def _parity3_weights(w3):
    _, _, cin, cout = w3.shape
    w3 = w3.astype(jnp.float32)
    t = jnp.arange(2)[:, None]
    s = jnp.arange(2)[None, :]
    halves = []
    for p in (0, 1):
        dj = 2 * t + s - p
        valid = (dj >= 0) & (dj <= 2)
        wd = w3[:, jnp.clip(dj, 0, 2), :, :]
        wd = jnp.where(valid[None, :, :, None, None], wd, 0.0)
        halves.append(wd.reshape(12 * cin, cout))
    return jnp.concatenate(halves, axis=-1).astype(jnp.bfloat16)


def _conv3x3_pool(x2, w3, b, *, rb, wbv):
    n, h, wp2, cin2 = x2.shape
    cout = w3.shape[-1]
    hp = (h - 2) // 2
    w_r = _parity3_weights(w3)
    b_r = b.astype(jnp.float32).reshape(1, cout)
    return pl.pallas_call(
        functools.partial(_c3x3_kernel, rb=rb, wbv=wbv),
        out_shape=jax.ShapeDtypeStruct((n, hp, wp2, cout), jnp.bfloat16),
        grid_spec=pltpu.PrefetchScalarGridSpec(
            num_scalar_prefetch=0,
            grid=(n, hp // rb),
            in_specs=[
                pl.BlockSpec((1, h, wp2, cin2), lambda i, r: (i, 0, 0, 0)),
                pl.BlockSpec((6 * cin2, 2 * cout), lambda i, r: (0, 0)),
                pl.BlockSpec((1, cout), lambda i, r: (0, 0)),
            ],
            out_specs=pl.BlockSpec((1, rb, wp2, cout), lambda i, r: (i, r, 0, 0)),
        ),
        compiler_params=pltpu.CompilerParams(
            dimension_semantics=("parallel", "arbitrary"),
            vmem_limit_bytes=_VMEM_LIMIT),
    )(x2, w_r, b_r)


def _fc_head_kernel(x_ref, w1_ref, b1_ref, w2_ref, b2_ref, w3_ref, b3_ref,
                    o_ref):
    h = jnp.dot(x_ref[...].astype(jnp.float32), w1_ref[...],
                preferred_element_type=jnp.float32) + b1_ref[...]
    h = jnp.maximum(h, 0.0)
    h = jnp.dot(h, w2_ref[...], preferred_element_type=jnp.float32) + b2_ref[...]
    h = jnp.maximum(h, 0.0)
    o = jnp.dot(h, w3_ref[...], preferred_element_type=jnp.float32) + b3_ref[...]
    o_ref[...] = o


def _conv1_weights(conv1_w):
    w5 = conv1_w.astype(jnp.float32)
    t = jnp.arange(3)[:, None]
    s = jnp.arange(2)[None, :]
    halves = []
    for p in (0, 1):
        dj = 2 * t + s - p
        valid = (dj >= 0) & (dj <= 4)
        wd = w5[:, jnp.clip(dj, 0, 4), :, :]
        wd = jnp.where(valid[None, :, :, None, None], wd, 0.0)
        halves.append(wd.reshape(90, 64))
    return jnp.concatenate(halves, axis=-1).astype(jnp.bfloat16)


def kernel(x_nchw, conv1_w, conv1_b, conv2_w, conv2_b, conv3_w, conv3_b,
           fc1_w, fc1_b, fc2_w, fc2_b, fc3_w, fc3_b):
    n = x_nchw.shape[0]

    xt = jnp.transpose(x_nchw, (0, 2, 3, 1)).astype(jnp.bfloat16)
    xp2 = xt.reshape(n, 224, 112, 6)
    xp2 = jnp.pad(xp2, ((0, 0), (0, 0), (0, 2), (0, 0)))
    xp = jnp.concatenate([xp2[:, :, t:t + 112, :] for t in range(3)],
                         axis=-1)
    w1 = _conv1_weights(conv1_w)
    b1 = conv1_b.astype(jnp.float32).reshape(1, 64)

    x = _conv1_pool(xp, w1, b1, rb=55)
    x = x.reshape(n, 110, 64, 128)
    x = _conv3x3_pool(x, conv2_w, conv2_b, rb=27, wbv=54)
    x = x.reshape(n, 54, 32, 384)
    x = _conv3x3_pool(x, conv3_w, conv3_b, rb=26, wbv=26)
    x = x[:, :, :26, :]

    w1p = fc1_w.reshape(16, 26, 26, 120).transpose(1, 2, 0, 3).reshape(10816, 120)
    xf = x.reshape(n, 10816)

    return pl.pallas_call(
        _fc_head_kernel,
        out_shape=jax.ShapeDtypeStruct((n, fc3_w.shape[1]), jnp.float32),
        compiler_params=pltpu.CompilerParams(vmem_limit_bytes=_VMEM_LIMIT),
    )(xf, w1p, fc1_b.reshape(1, -1),
      fc2_w, fc2_b.reshape(1, -1),
      fc3_w, fc3_b.reshape(1, -1))
